# baseline (speedup 1.0000x reference)
import functools

import jax
import jax.numpy as jnp
from jax import lax
from jax.experimental import pallas as pl
from jax.experimental.pallas import tpu as pltpu
from jax.experimental.pallas import tpu_sc as plsc

K = 20
KP = 32
EPS = 1e-5
N = 10000
NP = 10240
NPAD2 = 10016
ROWS_PER_W = NPAD2 // 32
BRT = 256
CH = 32
JD = 6
BE = 200
PADVAL = 1.0e6
GG = 8


def _lrelu(y):
    return jnp.where(y > 0, y, 0.2 * y)


def _dist_topk_kernel(xr_ref, xall_ref, o_ref):
    xr = xr_ref[...]
    xall = xall_ref[...]
    xxr = jnp.sum(xr * xr, axis=1)
    xxc = jnp.sum(xall * xall, axis=1)
    g = lax.dot_general(xall, xr, (((1,), (1,)), ((), ())),
                        preferred_element_type=jnp.float32)
    w = xxc[:, None] - 2.0 * g + xxr[None, :]
    NCH = NP // CH
    wm = w.reshape(NCH, CH, BRT)
    ich = lax.broadcasted_iota(jnp.int32, (NCH, CH, BRT), 1)
    chbase = lax.broadcasted_iota(jnp.int32, (NCH, BRT), 0) * CH
    Vs, Js = [], []
    for _ in range(JD):
        mv = jnp.min(wm, axis=1)
        cv = jnp.where(wm == mv[:, None, :], ich, CH)
        jv = jnp.min(cv, axis=1)
        Vs.append(mv)
        Js.append(jv + chbase)
        wm = jnp.where(ich == jv[:, None, :], jnp.inf, wm)
    V = jnp.concatenate(Vs, axis=0)
    J = jnp.concatenate(Js, axis=0)
    cols = []
    for _ in range(K):
        m = jnp.min(V, axis=0)
        cand = jnp.where(V == m[None, :], J, NP)
        jsel = jnp.min(cand, axis=0)
        cols.append(jsel)
        V = jnp.where(J == jsel[None, :], jnp.inf, V)
    for _ in range(K, KP):
        cols.append(cols[-1])
    o_ref[...] = jnp.stack(cols, axis=0)


def _dist_topk(x_pad):
    CP = x_pad.shape[1]
    return pl.pallas_call(
        _dist_topk_kernel,
        grid=(NP // BRT,),
        in_specs=[
            pl.BlockSpec((BRT, CP), lambda i: (i, 0)),
            pl.BlockSpec((NP, CP), lambda i: (0, 0)),
        ],
        out_specs=pl.BlockSpec((KP, BRT), lambda i: (0, i)),
        out_shape=jax.ShapeDtypeStruct((KP, NP), jnp.int32),
    )(x_pad, x_pad)


def _make_sc_gather(CP):
    mesh = plsc.VectorSubcoreMesh(core_axis_name="c", subcore_axis_name="s")

    @functools.partial(
        pl.kernel,
        out_type=[
            jax.ShapeDtypeStruct((NPAD2, 16, CP), jnp.float32),
            jax.ShapeDtypeStruct((NPAD2, 16, CP), jnp.float32),
        ],
        mesh=mesh,
        scratch_types=[
            pltpu.VMEM((ROWS_PER_W * KP,), jnp.int32),
            pltpu.VMEM((2 * GG, 16, CP), jnp.float32),
            pltpu.SemaphoreType.DMA,
            pltpu.SemaphoreType.DMA,
        ],
    )
    def sc_gather(idx_hbm, x_hbm, na_hbm, nb_hbm, idxv, slots, gsem, wsem):
        wid = lax.axis_index("s") * 2 + lax.axis_index("c")
        base = wid * ROWS_PER_W
        pltpu.sync_copy(idx_hbm.at[pl.ds(base * KP, ROWS_PER_W * KP)], idxv)

        def do_group(r0, ng):
            gets = []
            for u in range(ng):
                iva = idxv[pl.ds((r0 + u) * KP, 16)]
                ivb = idxv[pl.ds((r0 + u) * KP + 16, 16)]
                gets.append(pltpu.async_copy(x_hbm.at[iva], slots.at[2 * u],
                                             gsem))
                gets.append(pltpu.async_copy(x_hbm.at[ivb],
                                             slots.at[2 * u + 1], gsem))
            puts = []
            for u in range(ng):
                r = base + r0 + u
                gets[2 * u].wait()
                gets[2 * u + 1].wait()
                puts.append(pltpu.async_copy(slots.at[2 * u], na_hbm.at[r],
                                             wsem))
                puts.append(pltpu.async_copy(slots.at[2 * u + 1],
                                             nb_hbm.at[r], wsem))
            for p in puts:
                p.wait()

        def group_body(gi, carry):
            do_group(gi * GG, GG)
            return carry

        lax.fori_loop(0, ROWS_PER_W // GG, group_body, 0)
        do_group((ROWS_PER_W // GG) * GG, ROWS_PER_W % GG)

    return sc_gather


def _edge_kernel(na_ref, nb_ref, x_ref, wt_ref, s_ref, b_ref, o_ref):
    x = x_ref[...]
    C = x.shape[1]
    na = na_ref[...][:, :, :C]
    nb = nb_ref[...][:, :4, :C]
    neigh = jnp.concatenate([na, nb], axis=1)
    xc = jnp.broadcast_to(x[:, None, :], neigh.shape)
    f = jnp.concatenate([neigh - xc, xc], axis=-1)
    f2 = f.reshape(BE * K, 2 * C)
    y = jnp.dot(f2, wt_ref[...], preferred_element_type=jnp.float32)
    y = y * s_ref[...] + b_ref[...]
    y = _lrelu(y)
    o_ref[...] = jnp.max(y.reshape(BE, K, 64), axis=1)


def _edge_conv(na, nb, x, W, g, b):
    C = x.shape[1]
    CP = na.shape[2]
    s = (g / jnp.sqrt(1.0 + EPS)).reshape(1, 64)
    bb = b.reshape(1, 64)
    return pl.pallas_call(
        _edge_kernel,
        grid=(N // BE,),
        in_specs=[
            pl.BlockSpec((BE, 16, CP), lambda i: (i, 0, 0)),
            pl.BlockSpec((BE, 16, CP), lambda i: (i, 0, 0)),
            pl.BlockSpec((BE, C), lambda i: (i, 0)),
            pl.BlockSpec((2 * C, 64), lambda i: (0, 0)),
            pl.BlockSpec((1, 64), lambda i: (0, 0)),
            pl.BlockSpec((1, 64), lambda i: (0, 0)),
        ],
        out_specs=pl.BlockSpec((BE, 64), lambda i: (i, 0)),
        out_shape=jax.ShapeDtypeStruct((N, 64), jnp.float32),
    )(na, nb, x, W.T, s, bb)


def _edge_block(x, W, g, b):
    C = x.shape[1]
    CP = 128
    x_pad = jnp.full((NP, CP), PADVAL, jnp.float32)
    x_pad = x_pad.at[:, C:].set(0.0)
    x_pad = x_pad.at[:N, :C].set(x)
    idx = _dist_topk(x_pad).T
    idx_flat = idx[:NPAD2].reshape(NPAD2 * KP)
    na, nb = _make_sc_gather(CP)(idx_flat, x_pad)
    return _edge_conv(na, nb, x, W, g, b)


def _head_kernel(xc_ref, w5_ref, s5_ref, b5_ref, wf_ref, sf_ref, bf_ref,
                 ws_ref, bs_ref, o_ref):
    y = jnp.dot(xc_ref[...], w5_ref[...], preferred_element_type=jnp.float32)
    y = _lrelu(y * s5_ref[...] + b5_ref[...])
    y = jnp.dot(y, wf_ref[...], preferred_element_type=jnp.float32)
    y = y * sf_ref[...] + bf_ref[...]
    y = jnp.dot(y, ws_ref[...], preferred_element_type=jnp.float32)
    o_ref[...] = y + bs_ref[...]


def _head(xcat, W5, g5, b5, Wf, gout, bout, Ws, bs):
    s5 = (g5 / jnp.sqrt(1.0 + EPS)).reshape(1, 1024)
    sf = (gout / jnp.sqrt(1.0 + EPS)).reshape(1, 256)
    return pl.pallas_call(
        _head_kernel,
        grid=(N // BE,),
        in_specs=[
            pl.BlockSpec((BE, 256), lambda i: (i, 0)),
            pl.BlockSpec((256, 1024), lambda i: (0, 0)),
            pl.BlockSpec((1, 1024), lambda i: (0, 0)),
            pl.BlockSpec((1, 1024), lambda i: (0, 0)),
            pl.BlockSpec((1024, 256), lambda i: (0, 0)),
            pl.BlockSpec((1, 256), lambda i: (0, 0)),
            pl.BlockSpec((1, 256), lambda i: (0, 0)),
            pl.BlockSpec((256, 20), lambda i: (0, 0)),
            pl.BlockSpec((1, 20), lambda i: (0, 0)),
        ],
        out_specs=pl.BlockSpec((BE, 20), lambda i: (i, 0)),
        out_shape=jax.ShapeDtypeStruct((N, 20), jnp.float32),
    )(xcat, W5.T, s5, b5.reshape(1, 1024), Wf.T, sf, bout.reshape(1, 256),
      Ws.T, bs.reshape(1, 20))


def kernel(points, features, W1, g1, b1, W2, g2, b2, W3, g3, b3, W4, g4, b4, W5, g5, b5, Wf, gout, bout, Ws, bs):
    x = jnp.concatenate([points, features], axis=1)
    x1 = _edge_block(x, W1, g1, b1)
    x2 = _edge_block(x1, W2, g2, b2)
    x3 = _edge_block(x2, W3, g3, b3)
    x4 = _edge_block(x3, W4, g4, b4)
    xcat = jnp.concatenate([x1, x2, x3, x4], axis=1)
    return _head(xcat, W5, g5, b5, Wf, gout, bout, Ws, bs)


# speedup vs baseline: 1.0986x; 1.0986x over previous
import functools

import jax
import jax.numpy as jnp
from jax import lax
from jax.experimental import pallas as pl
from jax.experimental.pallas import tpu as pltpu
from jax.experimental.pallas import tpu_sc as plsc

K = 20
KP = 32
EPS = 1e-5
N = 10000
NP = 10240
NPAD2 = 10016
ROWS_PER_W = NPAD2 // 32
BRT = 128
CH = 32
JD = 6
BE = 200
PADVAL = 1.0e6
GG = 8


def _lrelu(y):
    return jnp.where(y > 0, y, 0.2 * y)


def _dist_topk_kernel(xr_ref, xall_ref, o_ref):
    xr = xr_ref[...]
    xall = xall_ref[...]
    xxr = jnp.sum(xr * xr, axis=1)
    xxc = jnp.sum(xall * xall, axis=1)
    g = lax.dot_general(xall, xr, (((1,), (1,)), ((), ())),
                        preferred_element_type=jnp.float32)
    w = xxc[:, None] - 2.0 * g + xxr[None, :]
    NCH = NP // CH
    wm = w.reshape(NCH, CH, BRT)
    ich = lax.broadcasted_iota(jnp.int32, (NCH, CH, BRT), 1)
    chbase = lax.broadcasted_iota(jnp.int32, (NCH, BRT), 0) * CH
    Vs, Js = [], []
    for _ in range(JD):
        mv = jnp.min(wm, axis=1)
        cv = jnp.where(wm == mv[:, None, :], ich, CH)
        jv = jnp.min(cv, axis=1)
        Vs.append(mv)
        Js.append(jv + chbase)
        wm = jnp.where(ich == jv[:, None, :], jnp.inf, wm)
    V = jnp.concatenate(Vs, axis=0)
    J = jnp.concatenate(Js, axis=0)
    cols = []
    for _ in range(K):
        m = jnp.min(V, axis=0)
        cand = jnp.where(V == m[None, :], J, NP)
        jsel = jnp.min(cand, axis=0)
        cols.append(jsel)
        V = jnp.where(J == jsel[None, :], jnp.inf, V)
    for _ in range(K, KP):
        cols.append(cols[-1])
    o_ref[...] = jnp.stack(cols, axis=0)


def _dist_topk(x_pad):
    CP = x_pad.shape[1]
    return pl.pallas_call(
        _dist_topk_kernel,
        grid=(NP // BRT,),
        in_specs=[
            pl.BlockSpec((BRT, CP), lambda i: (i, 0)),
            pl.BlockSpec((NP, CP), lambda i: (0, 0)),
        ],
        out_specs=pl.BlockSpec((KP, BRT), lambda i: (0, i)),
        out_shape=jax.ShapeDtypeStruct((KP, NP), jnp.int32),
    )(x_pad, x_pad)


def _make_sc_gather(CP):
    mesh = plsc.VectorSubcoreMesh(core_axis_name="c", subcore_axis_name="s")

    @functools.partial(
        pl.kernel,
        out_type=[
            jax.ShapeDtypeStruct((NPAD2, 16, CP), jnp.float32),
            jax.ShapeDtypeStruct((NPAD2, 16, CP), jnp.float32),
        ],
        mesh=mesh,
        scratch_types=[
            pltpu.VMEM((ROWS_PER_W * KP,), jnp.int32),
            pltpu.VMEM((2 * GG, 16, CP), jnp.float32),
            pltpu.SemaphoreType.DMA,
            pltpu.SemaphoreType.DMA,
        ],
    )
    def sc_gather(idx_hbm, x_hbm, na_hbm, nb_hbm, idxv, slots, gsem, wsem):
        wid = lax.axis_index("s") * 2 + lax.axis_index("c")
        base = wid * ROWS_PER_W
        pltpu.sync_copy(idx_hbm.at[pl.ds(base * KP, ROWS_PER_W * KP)], idxv)

        def do_group(r0, ng):
            gets = []
            for u in range(ng):
                iva = idxv[pl.ds((r0 + u) * KP, 16)]
                ivb = idxv[pl.ds((r0 + u) * KP + 16, 16)]
                gets.append(pltpu.async_copy(x_hbm.at[iva], slots.at[2 * u],
                                             gsem))
                gets.append(pltpu.async_copy(x_hbm.at[ivb],
                                             slots.at[2 * u + 1], gsem))
            puts = []
            for u in range(ng):
                r = base + r0 + u
                gets[2 * u].wait()
                gets[2 * u + 1].wait()
                puts.append(pltpu.async_copy(slots.at[2 * u], na_hbm.at[r],
                                             wsem))
                puts.append(pltpu.async_copy(slots.at[2 * u + 1],
                                             nb_hbm.at[r], wsem))
            for p in puts:
                p.wait()

        def group_body(gi, carry):
            do_group(gi * GG, GG)
            return carry

        lax.fori_loop(0, ROWS_PER_W // GG, group_body, 0)
        do_group((ROWS_PER_W // GG) * GG, ROWS_PER_W % GG)

    return sc_gather


def _edge_kernel(na_ref, nb_ref, x_ref, wt_ref, s_ref, b_ref, o_ref):
    x = x_ref[...]
    C = x.shape[1]
    na = na_ref[...][:, :, :C]
    nb = nb_ref[...][:, :4, :C]
    neigh = jnp.concatenate([na, nb], axis=1)
    xc = jnp.broadcast_to(x[:, None, :], neigh.shape)
    f = jnp.concatenate([neigh - xc, xc], axis=-1)
    f2 = f.reshape(BE * K, 2 * C)
    y = jnp.dot(f2, wt_ref[...], preferred_element_type=jnp.float32)
    y = y * s_ref[...] + b_ref[...]
    y = _lrelu(y)
    o_ref[...] = jnp.max(y.reshape(BE, K, 64), axis=1)


def _edge_conv(na, nb, x, W, g, b):
    C = x.shape[1]
    CP = na.shape[2]
    s = (g / jnp.sqrt(1.0 + EPS)).reshape(1, 64)
    bb = b.reshape(1, 64)
    return pl.pallas_call(
        _edge_kernel,
        grid=(N // BE,),
        in_specs=[
            pl.BlockSpec((BE, 16, CP), lambda i: (i, 0, 0)),
            pl.BlockSpec((BE, 16, CP), lambda i: (i, 0, 0)),
            pl.BlockSpec((BE, C), lambda i: (i, 0)),
            pl.BlockSpec((2 * C, 64), lambda i: (0, 0)),
            pl.BlockSpec((1, 64), lambda i: (0, 0)),
            pl.BlockSpec((1, 64), lambda i: (0, 0)),
        ],
        out_specs=pl.BlockSpec((BE, 64), lambda i: (i, 0)),
        out_shape=jax.ShapeDtypeStruct((N, 64), jnp.float32),
    )(na, nb, x, W.T, s, bb)


def _edge_block(x, W, g, b):
    C = x.shape[1]
    CP = 128
    x_pad = jnp.full((NP, CP), PADVAL, jnp.float32)
    x_pad = x_pad.at[:, C:].set(0.0)
    x_pad = x_pad.at[:N, :C].set(x)
    idx = _dist_topk(x_pad).T
    idx_flat = idx[:NPAD2].reshape(NPAD2 * KP)
    na, nb = _make_sc_gather(CP)(idx_flat, x_pad)
    return _edge_conv(na, nb, x, W, g, b)


def _head_kernel(xc_ref, w5_ref, s5_ref, b5_ref, wf_ref, sf_ref, bf_ref,
                 ws_ref, bs_ref, o_ref):
    y = jnp.dot(xc_ref[...], w5_ref[...], preferred_element_type=jnp.float32)
    y = _lrelu(y * s5_ref[...] + b5_ref[...])
    y = jnp.dot(y, wf_ref[...], preferred_element_type=jnp.float32)
    y = y * sf_ref[...] + bf_ref[...]
    y = jnp.dot(y, ws_ref[...], preferred_element_type=jnp.float32)
    o_ref[...] = y + bs_ref[...]


def _head(xcat, W5, g5, b5, Wf, gout, bout, Ws, bs):
    s5 = (g5 / jnp.sqrt(1.0 + EPS)).reshape(1, 1024)
    sf = (gout / jnp.sqrt(1.0 + EPS)).reshape(1, 256)
    return pl.pallas_call(
        _head_kernel,
        grid=(N // BE,),
        in_specs=[
            pl.BlockSpec((BE, 256), lambda i: (i, 0)),
            pl.BlockSpec((256, 1024), lambda i: (0, 0)),
            pl.BlockSpec((1, 1024), lambda i: (0, 0)),
            pl.BlockSpec((1, 1024), lambda i: (0, 0)),
            pl.BlockSpec((1024, 256), lambda i: (0, 0)),
            pl.BlockSpec((1, 256), lambda i: (0, 0)),
            pl.BlockSpec((1, 256), lambda i: (0, 0)),
            pl.BlockSpec((256, 20), lambda i: (0, 0)),
            pl.BlockSpec((1, 20), lambda i: (0, 0)),
        ],
        out_specs=pl.BlockSpec((BE, 20), lambda i: (i, 0)),
        out_shape=jax.ShapeDtypeStruct((N, 20), jnp.float32),
    )(xcat, W5.T, s5, b5.reshape(1, 1024), Wf.T, sf, bout.reshape(1, 256),
      Ws.T, bs.reshape(1, 20))


def kernel(points, features, W1, g1, b1, W2, g2, b2, W3, g3, b3, W4, g4, b4, W5, g5, b5, Wf, gout, bout, Ws, bs):
    x = jnp.concatenate([points, features], axis=1)
    x1 = _edge_block(x, W1, g1, b1)
    x2 = _edge_block(x1, W2, g2, b2)
    x3 = _edge_block(x2, W3, g3, b3)
    x4 = _edge_block(x3, W4, g4, b4)
    xcat = jnp.concatenate([x1, x2, x3, x4], axis=1)
    return _head(xcat, W5, g5, b5, Wf, gout, bout, Ws, bs)


# speedup vs baseline: 1.4814x; 1.3484x over previous
import functools

import jax
import jax.numpy as jnp
from jax import lax
from jax.experimental import pallas as pl
from jax.experimental.pallas import tpu as pltpu
from jax.experimental.pallas import tpu_sc as plsc

K = 20
KP = 32
EPS = 1e-5
N = 10000
NP = 10240
NPAD2 = 10016
ROWS_PER_W = NPAD2 // 32
BRT = 128
CH = 32
JD = 4
BE = 200
PADVAL = 1.0e6
GG = 8


def _lrelu(y):
    return jnp.where(y > 0, y, 0.2 * y)


def _dist_topk_kernel(xr_ref, xall_ref, o_ref):
    xr = xr_ref[...]
    xall = xall_ref[...]
    xxr = jnp.sum(xr * xr, axis=1)
    xxc = jnp.sum(xall * xall, axis=1)
    g = lax.dot_general(xall, xr, (((1,), (1,)), ((), ())),
                        preferred_element_type=jnp.float32)
    w = xxc[:, None] - 2.0 * g + xxr[None, :]
    NCH = NP // CH
    wm = w.reshape(NCH, CH, BRT)
    ich = lax.broadcasted_iota(jnp.int32, (NCH, CH, BRT), 1)
    chbase = lax.broadcasted_iota(jnp.int32, (NCH, BRT), 0) * CH
    Vs, Js = [], []
    for _ in range(JD):
        mv = jnp.min(wm, axis=1)
        cv = jnp.where(wm == mv[:, None, :], ich, CH)
        jv = jnp.min(cv, axis=1)
        Vs.append(mv)
        Js.append(jv + chbase)
        wm = jnp.where(ich == jv[:, None, :], jnp.inf, wm)
    V = jnp.concatenate(Vs, axis=0)
    J = jnp.concatenate(Js, axis=0)
    cols = []
    for _ in range(K):
        m = jnp.min(V, axis=0)
        cand = jnp.where(V == m[None, :], J, NP)
        jsel = jnp.min(cand, axis=0)
        cols.append(jsel)
        V = jnp.where(J == jsel[None, :], jnp.inf, V)
    for _ in range(K, KP):
        cols.append(cols[-1])
    o_ref[...] = jnp.stack(cols, axis=0)


def _dist_topk(x_pad):
    CP = x_pad.shape[1]
    return pl.pallas_call(
        _dist_topk_kernel,
        grid=(NP // BRT,),
        in_specs=[
            pl.BlockSpec((BRT, CP), lambda i: (i, 0)),
            pl.BlockSpec((NP, CP), lambda i: (0, 0)),
        ],
        out_specs=pl.BlockSpec((KP, BRT), lambda i: (0, i)),
        out_shape=jax.ShapeDtypeStruct((KP, NP), jnp.int32),
    )(x_pad, x_pad)


def _make_sc_gather(CP):
    mesh = plsc.VectorSubcoreMesh(core_axis_name="c", subcore_axis_name="s")

    @functools.partial(
        pl.kernel,
        out_type=[
            jax.ShapeDtypeStruct((NPAD2, 16, CP), jnp.float32),
            jax.ShapeDtypeStruct((NPAD2, 16, CP), jnp.float32),
        ],
        mesh=mesh,
        scratch_types=[
            pltpu.VMEM((ROWS_PER_W * KP,), jnp.int32),
            pltpu.VMEM((2 * GG, 16, CP), jnp.float32),
            pltpu.SemaphoreType.DMA,
            pltpu.SemaphoreType.DMA,
        ],
    )
    def sc_gather(idx_hbm, x_hbm, na_hbm, nb_hbm, idxv, slots, gsem, wsem):
        wid = lax.axis_index("s") * 2 + lax.axis_index("c")
        base = wid * ROWS_PER_W
        pltpu.sync_copy(idx_hbm.at[pl.ds(base * KP, ROWS_PER_W * KP)], idxv)

        def do_group(r0, ng):
            gets = []
            for u in range(ng):
                iva = idxv[pl.ds((r0 + u) * KP, 16)]
                ivb = idxv[pl.ds((r0 + u) * KP + 16, 16)]
                gets.append(pltpu.async_copy(x_hbm.at[iva], slots.at[2 * u],
                                             gsem))
                gets.append(pltpu.async_copy(x_hbm.at[ivb],
                                             slots.at[2 * u + 1], gsem))
            puts = []
            for u in range(ng):
                r = base + r0 + u
                gets[2 * u].wait()
                gets[2 * u + 1].wait()
                puts.append(pltpu.async_copy(slots.at[2 * u], na_hbm.at[r],
                                             wsem))
                puts.append(pltpu.async_copy(slots.at[2 * u + 1],
                                             nb_hbm.at[r], wsem))
            for p in puts:
                p.wait()

        def group_body(gi, carry):
            do_group(gi * GG, GG)
            return carry

        lax.fori_loop(0, ROWS_PER_W // GG, group_body, 0)
        do_group((ROWS_PER_W // GG) * GG, ROWS_PER_W % GG)

    return sc_gather


def _edge_kernel(na_ref, nb_ref, x_ref, wt_ref, s_ref, b_ref, o_ref):
    x = x_ref[...]
    C = x.shape[1]
    na = na_ref[...][:, :, :C]
    nb = nb_ref[...][:, :4, :C]
    neigh = jnp.concatenate([na, nb], axis=1)
    xc = jnp.broadcast_to(x[:, None, :], neigh.shape)
    f = jnp.concatenate([neigh - xc, xc], axis=-1)
    f2 = f.reshape(BE * K, 2 * C)
    y = jnp.dot(f2, wt_ref[...], preferred_element_type=jnp.float32)
    y = y * s_ref[...] + b_ref[...]
    y = _lrelu(y)
    o_ref[...] = jnp.max(y.reshape(BE, K, 64), axis=1)


def _edge_conv(na, nb, x, W, g, b):
    C = x.shape[1]
    CP = na.shape[2]
    s = (g / jnp.sqrt(1.0 + EPS)).reshape(1, 64)
    bb = b.reshape(1, 64)
    return pl.pallas_call(
        _edge_kernel,
        grid=(N // BE,),
        in_specs=[
            pl.BlockSpec((BE, 16, CP), lambda i: (i, 0, 0)),
            pl.BlockSpec((BE, 16, CP), lambda i: (i, 0, 0)),
            pl.BlockSpec((BE, C), lambda i: (i, 0)),
            pl.BlockSpec((2 * C, 64), lambda i: (0, 0)),
            pl.BlockSpec((1, 64), lambda i: (0, 0)),
            pl.BlockSpec((1, 64), lambda i: (0, 0)),
        ],
        out_specs=pl.BlockSpec((BE, 64), lambda i: (i, 0)),
        out_shape=jax.ShapeDtypeStruct((N, 64), jnp.float32),
    )(na, nb, x, W.T, s, bb)


def _edge_block(x, W, g, b):
    C = x.shape[1]
    CP = 128
    x_pad = jnp.full((NP, CP), PADVAL, jnp.float32)
    x_pad = x_pad.at[:, C:].set(0.0)
    x_pad = x_pad.at[:N, :C].set(x)
    idx = _dist_topk(x_pad).T
    idx_flat = idx[:NPAD2].reshape(NPAD2 * KP)
    na, nb = _make_sc_gather(CP)(idx_flat, x_pad)
    return _edge_conv(na, nb, x, W, g, b)


def _head_kernel(xc_ref, w5_ref, s5_ref, b5_ref, wf_ref, sf_ref, bf_ref,
                 ws_ref, bs_ref, o_ref):
    y = jnp.dot(xc_ref[...], w5_ref[...], preferred_element_type=jnp.float32)
    y = _lrelu(y * s5_ref[...] + b5_ref[...])
    y = jnp.dot(y, wf_ref[...], preferred_element_type=jnp.float32)
    y = y * sf_ref[...] + bf_ref[...]
    y = jnp.dot(y, ws_ref[...], preferred_element_type=jnp.float32)
    o_ref[...] = y + bs_ref[...]


def _head(xcat, W5, g5, b5, Wf, gout, bout, Ws, bs):
    s5 = (g5 / jnp.sqrt(1.0 + EPS)).reshape(1, 1024)
    sf = (gout / jnp.sqrt(1.0 + EPS)).reshape(1, 256)
    return pl.pallas_call(
        _head_kernel,
        grid=(N // BE,),
        in_specs=[
            pl.BlockSpec((BE, 256), lambda i: (i, 0)),
            pl.BlockSpec((256, 1024), lambda i: (0, 0)),
            pl.BlockSpec((1, 1024), lambda i: (0, 0)),
            pl.BlockSpec((1, 1024), lambda i: (0, 0)),
            pl.BlockSpec((1024, 256), lambda i: (0, 0)),
            pl.BlockSpec((1, 256), lambda i: (0, 0)),
            pl.BlockSpec((1, 256), lambda i: (0, 0)),
            pl.BlockSpec((256, 20), lambda i: (0, 0)),
            pl.BlockSpec((1, 20), lambda i: (0, 0)),
        ],
        out_specs=pl.BlockSpec((BE, 20), lambda i: (i, 0)),
        out_shape=jax.ShapeDtypeStruct((N, 20), jnp.float32),
    )(xcat, W5.T, s5, b5.reshape(1, 1024), Wf.T, sf, bout.reshape(1, 256),
      Ws.T, bs.reshape(1, 20))


def kernel(points, features, W1, g1, b1, W2, g2, b2, W3, g3, b3, W4, g4, b4, W5, g5, b5, Wf, gout, bout, Ws, bs):
    x = jnp.concatenate([points, features], axis=1)
    x1 = _edge_block(x, W1, g1, b1)
    x2 = _edge_block(x1, W2, g2, b2)
    x3 = _edge_block(x2, W3, g3, b3)
    x4 = _edge_block(x3, W4, g4, b4)
    xcat = jnp.concatenate([x1, x2, x3, x4], axis=1)
    return _head(xcat, W5, g5, b5, Wf, gout, bout, Ws, bs)


# speedup vs baseline: 1.8035x; 1.2174x over previous
import functools

import jax
import jax.numpy as jnp
from jax import lax
from jax.experimental import pallas as pl
from jax.experimental.pallas import tpu as pltpu
from jax.experimental.pallas import tpu_sc as plsc

K = 20
KP = 32
EPS = 1e-5
N = 10000
NP = 10240
NPAD2 = 10016
ROWS_PER_W = NPAD2 // 32
BRT = 128
CH = 32
JD = 3
BE = 200
PADVAL = 1.0e6
GG = 8


def _lrelu(y):
    return jnp.where(y > 0, y, 0.2 * y)


def _dist_topk_kernel(xr_ref, xall_ref, o_ref):
    xr = xr_ref[...]
    xall = xall_ref[...]
    xxr = jnp.sum(xr * xr, axis=1)
    xxc = jnp.sum(xall * xall, axis=1)
    g = lax.dot_general(xall, xr, (((1,), (1,)), ((), ())),
                        preferred_element_type=jnp.float32)
    w = xxc[:, None] - 2.0 * g + xxr[None, :]
    NCH = NP // CH
    wm = w.reshape(NCH, CH, BRT)
    ich = lax.broadcasted_iota(jnp.int32, (NCH, CH, BRT), 1)
    chbase = lax.broadcasted_iota(jnp.int32, (NCH, BRT), 0) * CH
    Vs, Js = [], []
    for _ in range(JD):
        mv = jnp.min(wm, axis=1)
        cv = jnp.where(wm == mv[:, None, :], ich, CH)
        jv = jnp.min(cv, axis=1)
        Vs.append(mv)
        Js.append(jv + chbase)
        wm = jnp.where(ich == jv[:, None, :], jnp.inf, wm)
    V = jnp.concatenate(Vs, axis=0)
    J = jnp.concatenate(Js, axis=0)
    cols = []
    for _ in range(K):
        m = jnp.min(V, axis=0)
        cand = jnp.where(V == m[None, :], J, NP)
        jsel = jnp.min(cand, axis=0)
        cols.append(jsel)
        V = jnp.where(J == jsel[None, :], jnp.inf, V)
    for _ in range(K, KP):
        cols.append(cols[-1])
    o_ref[...] = jnp.stack(cols, axis=0)


def _dist_topk(x_pad):
    CP = x_pad.shape[1]
    return pl.pallas_call(
        _dist_topk_kernel,
        grid=(NP // BRT,),
        in_specs=[
            pl.BlockSpec((BRT, CP), lambda i: (i, 0)),
            pl.BlockSpec((NP, CP), lambda i: (0, 0)),
        ],
        out_specs=pl.BlockSpec((KP, BRT), lambda i: (0, i)),
        out_shape=jax.ShapeDtypeStruct((KP, NP), jnp.int32),
    )(x_pad, x_pad)


def _make_sc_gather(CP):
    mesh = plsc.VectorSubcoreMesh(core_axis_name="c", subcore_axis_name="s")

    @functools.partial(
        pl.kernel,
        out_type=[
            jax.ShapeDtypeStruct((NPAD2, 16, CP), jnp.float32),
            jax.ShapeDtypeStruct((NPAD2, 16, CP), jnp.float32),
        ],
        mesh=mesh,
        scratch_types=[
            pltpu.VMEM((ROWS_PER_W * KP,), jnp.int32),
            pltpu.VMEM((2 * GG, 16, CP), jnp.float32),
            pltpu.SemaphoreType.DMA,
            pltpu.SemaphoreType.DMA,
        ],
    )
    def sc_gather(idx_hbm, x_hbm, na_hbm, nb_hbm, idxv, slots, gsem, wsem):
        wid = lax.axis_index("s") * 2 + lax.axis_index("c")
        base = wid * ROWS_PER_W
        pltpu.sync_copy(idx_hbm.at[pl.ds(base * KP, ROWS_PER_W * KP)], idxv)

        def do_group(r0, ng):
            gets = []
            for u in range(ng):
                iva = idxv[pl.ds((r0 + u) * KP, 16)]
                ivb = idxv[pl.ds((r0 + u) * KP + 16, 16)]
                gets.append(pltpu.async_copy(x_hbm.at[iva], slots.at[2 * u],
                                             gsem))
                gets.append(pltpu.async_copy(x_hbm.at[ivb],
                                             slots.at[2 * u + 1], gsem))
            puts = []
            for u in range(ng):
                r = base + r0 + u
                gets[2 * u].wait()
                gets[2 * u + 1].wait()
                puts.append(pltpu.async_copy(slots.at[2 * u], na_hbm.at[r],
                                             wsem))
                puts.append(pltpu.async_copy(slots.at[2 * u + 1],
                                             nb_hbm.at[r], wsem))
            for p in puts:
                p.wait()

        def group_body(gi, carry):
            do_group(gi * GG, GG)
            return carry

        lax.fori_loop(0, ROWS_PER_W // GG, group_body, 0)
        do_group((ROWS_PER_W // GG) * GG, ROWS_PER_W % GG)

    return sc_gather


def _edge_kernel(na_ref, nb_ref, x_ref, wt_ref, s_ref, b_ref, o_ref):
    x = x_ref[...]
    C = x.shape[1]
    na = na_ref[...][:, :, :C]
    nb = nb_ref[...][:, :4, :C]
    neigh = jnp.concatenate([na, nb], axis=1)
    xc = jnp.broadcast_to(x[:, None, :], neigh.shape)
    f = jnp.concatenate([neigh - xc, xc], axis=-1)
    f2 = f.reshape(BE * K, 2 * C)
    y = jnp.dot(f2, wt_ref[...], preferred_element_type=jnp.float32)
    y = y * s_ref[...] + b_ref[...]
    y = _lrelu(y)
    o_ref[...] = jnp.max(y.reshape(BE, K, 64), axis=1)


def _edge_conv(na, nb, x, W, g, b):
    C = x.shape[1]
    CP = na.shape[2]
    s = (g / jnp.sqrt(1.0 + EPS)).reshape(1, 64)
    bb = b.reshape(1, 64)
    return pl.pallas_call(
        _edge_kernel,
        grid=(N // BE,),
        in_specs=[
            pl.BlockSpec((BE, 16, CP), lambda i: (i, 0, 0)),
            pl.BlockSpec((BE, 16, CP), lambda i: (i, 0, 0)),
            pl.BlockSpec((BE, C), lambda i: (i, 0)),
            pl.BlockSpec((2 * C, 64), lambda i: (0, 0)),
            pl.BlockSpec((1, 64), lambda i: (0, 0)),
            pl.BlockSpec((1, 64), lambda i: (0, 0)),
        ],
        out_specs=pl.BlockSpec((BE, 64), lambda i: (i, 0)),
        out_shape=jax.ShapeDtypeStruct((N, 64), jnp.float32),
    )(na, nb, x, W.T, s, bb)


def _edge_block(x, W, g, b):
    C = x.shape[1]
    CP = 128
    x_pad = jnp.full((NP, CP), PADVAL, jnp.float32)
    x_pad = x_pad.at[:, C:].set(0.0)
    x_pad = x_pad.at[:N, :C].set(x)
    idx = _dist_topk(x_pad).T
    idx_flat = idx[:NPAD2].reshape(NPAD2 * KP)
    na, nb = _make_sc_gather(CP)(idx_flat, x_pad)
    return _edge_conv(na, nb, x, W, g, b)


def _head_kernel(xc_ref, w5_ref, s5_ref, b5_ref, wf_ref, sf_ref, bf_ref,
                 ws_ref, bs_ref, o_ref):
    y = jnp.dot(xc_ref[...], w5_ref[...], preferred_element_type=jnp.float32)
    y = _lrelu(y * s5_ref[...] + b5_ref[...])
    y = jnp.dot(y, wf_ref[...], preferred_element_type=jnp.float32)
    y = y * sf_ref[...] + bf_ref[...]
    y = jnp.dot(y, ws_ref[...], preferred_element_type=jnp.float32)
    o_ref[...] = y + bs_ref[...]


def _head(xcat, W5, g5, b5, Wf, gout, bout, Ws, bs):
    s5 = (g5 / jnp.sqrt(1.0 + EPS)).reshape(1, 1024)
    sf = (gout / jnp.sqrt(1.0 + EPS)).reshape(1, 256)
    return pl.pallas_call(
        _head_kernel,
        grid=(N // BE,),
        in_specs=[
            pl.BlockSpec((BE, 256), lambda i: (i, 0)),
            pl.BlockSpec((256, 1024), lambda i: (0, 0)),
            pl.BlockSpec((1, 1024), lambda i: (0, 0)),
            pl.BlockSpec((1, 1024), lambda i: (0, 0)),
            pl.BlockSpec((1024, 256), lambda i: (0, 0)),
            pl.BlockSpec((1, 256), lambda i: (0, 0)),
            pl.BlockSpec((1, 256), lambda i: (0, 0)),
            pl.BlockSpec((256, 20), lambda i: (0, 0)),
            pl.BlockSpec((1, 20), lambda i: (0, 0)),
        ],
        out_specs=pl.BlockSpec((BE, 20), lambda i: (i, 0)),
        out_shape=jax.ShapeDtypeStruct((N, 20), jnp.float32),
    )(xcat, W5.T, s5, b5.reshape(1, 1024), Wf.T, sf, bout.reshape(1, 256),
      Ws.T, bs.reshape(1, 20))


def kernel(points, features, W1, g1, b1, W2, g2, b2, W3, g3, b3, W4, g4, b4, W5, g5, b5, Wf, gout, bout, Ws, bs):
    x = jnp.concatenate([points, features], axis=1)
    x1 = _edge_block(x, W1, g1, b1)
    x2 = _edge_block(x1, W2, g2, b2)
    x3 = _edge_block(x2, W3, g3, b3)
    x4 = _edge_block(x3, W4, g4, b4)
    xcat = jnp.concatenate([x1, x2, x3, x4], axis=1)
    return _head(xcat, W5, g5, b5, Wf, gout, bout, Ws, bs)


# speedup vs baseline: 1.8511x; 1.0264x over previous
import functools

import jax
import jax.numpy as jnp
from jax import lax
from jax.experimental import pallas as pl
from jax.experimental.pallas import tpu as pltpu
from jax.experimental.pallas import tpu_sc as plsc

K = 20
KP = 32
EPS = 1e-5
N = 10000
NP = 10240
NPAD2 = 10016
ROWS_PER_W = NPAD2 // 32
BRT = 128
CH = 32
JD = 3
BE = 200
PADVAL = 1.0e6
GG = 16


def _lrelu(y):
    return jnp.where(y > 0, y, 0.2 * y)


def _dist_topk_kernel(xr_ref, xall_ref, o_ref):
    xr = xr_ref[...]
    xall = xall_ref[...]
    xxr = jnp.sum(xr * xr, axis=1)
    xxc = jnp.sum(xall * xall, axis=1)
    g = lax.dot_general(xall, xr, (((1,), (1,)), ((), ())),
                        preferred_element_type=jnp.float32)
    w = xxc[:, None] - 2.0 * g + xxr[None, :]
    NCH = NP // CH
    wm = w.reshape(NCH, CH, BRT)
    ich = lax.broadcasted_iota(jnp.int32, (NCH, CH, BRT), 1)
    chbase = lax.broadcasted_iota(jnp.int32, (NCH, BRT), 0) * CH
    Vs, Js = [], []
    for _ in range(JD):
        mv = jnp.min(wm, axis=1)
        cv = jnp.where(wm == mv[:, None, :], ich, CH)
        jv = jnp.min(cv, axis=1)
        Vs.append(mv)
        Js.append(jv + chbase)
        wm = jnp.where(ich == jv[:, None, :], jnp.inf, wm)
    V = jnp.concatenate(Vs, axis=0)
    J = jnp.concatenate(Js, axis=0)
    cols = []
    for _ in range(K):
        m = jnp.min(V, axis=0)
        cand = jnp.where(V == m[None, :], J, NP)
        jsel = jnp.min(cand, axis=0)
        cols.append(jsel)
        V = jnp.where(J == jsel[None, :], jnp.inf, V)
    for _ in range(K, KP):
        cols.append(cols[-1])
    o_ref[...] = jnp.stack(cols, axis=0)


def _dist_topk(x_pad):
    CP = x_pad.shape[1]
    return pl.pallas_call(
        _dist_topk_kernel,
        grid=(NP // BRT,),
        in_specs=[
            pl.BlockSpec((BRT, CP), lambda i: (i, 0)),
            pl.BlockSpec((NP, CP), lambda i: (0, 0)),
        ],
        out_specs=pl.BlockSpec((KP, BRT), lambda i: (0, i)),
        out_shape=jax.ShapeDtypeStruct((KP, NP), jnp.int32),
    )(x_pad, x_pad)


def _make_sc_gather(CP):
    mesh = plsc.VectorSubcoreMesh(core_axis_name="c", subcore_axis_name="s")

    @functools.partial(
        pl.kernel,
        out_type=[
            jax.ShapeDtypeStruct((NPAD2, 16, CP), jnp.float32),
            jax.ShapeDtypeStruct((NPAD2, 16, CP), jnp.float32),
        ],
        mesh=mesh,
        scratch_types=[
            pltpu.VMEM((ROWS_PER_W * KP,), jnp.int32),
            pltpu.VMEM((2 * GG, 16, CP), jnp.float32),
            pltpu.SemaphoreType.DMA,
            pltpu.SemaphoreType.DMA,
        ],
    )
    def sc_gather(idx_hbm, x_hbm, na_hbm, nb_hbm, idxv, slots, gsem, wsem):
        wid = lax.axis_index("s") * 2 + lax.axis_index("c")
        base = wid * ROWS_PER_W
        pltpu.sync_copy(idx_hbm.at[pl.ds(base * KP, ROWS_PER_W * KP)], idxv)

        def do_group(r0, ng):
            gets = []
            for u in range(ng):
                iva = idxv[pl.ds((r0 + u) * KP, 16)]
                ivb = idxv[pl.ds((r0 + u) * KP + 16, 16)]
                gets.append(pltpu.async_copy(x_hbm.at[iva], slots.at[2 * u],
                                             gsem))
                gets.append(pltpu.async_copy(x_hbm.at[ivb],
                                             slots.at[2 * u + 1], gsem))
            puts = []
            for u in range(ng):
                r = base + r0 + u
                gets[2 * u].wait()
                gets[2 * u + 1].wait()
                puts.append(pltpu.async_copy(slots.at[2 * u], na_hbm.at[r],
                                             wsem))
                puts.append(pltpu.async_copy(slots.at[2 * u + 1],
                                             nb_hbm.at[r], wsem))
            for p in puts:
                p.wait()

        def group_body(gi, carry):
            do_group(gi * GG, GG)
            return carry

        lax.fori_loop(0, ROWS_PER_W // GG, group_body, 0)
        do_group((ROWS_PER_W // GG) * GG, ROWS_PER_W % GG)

    return sc_gather


def _edge_kernel(na_ref, nb_ref, x_ref, wt_ref, s_ref, b_ref, o_ref):
    x = x_ref[...]
    C = x.shape[1]
    na = na_ref[...][:, :, :C]
    nb = nb_ref[...][:, :4, :C]
    neigh = jnp.concatenate([na, nb], axis=1)
    xc = jnp.broadcast_to(x[:, None, :], neigh.shape)
    f = jnp.concatenate([neigh - xc, xc], axis=-1)
    f2 = f.reshape(BE * K, 2 * C)
    y = jnp.dot(f2, wt_ref[...], preferred_element_type=jnp.float32)
    y = y * s_ref[...] + b_ref[...]
    y = _lrelu(y)
    o_ref[...] = jnp.max(y.reshape(BE, K, 64), axis=1)


def _edge_conv(na, nb, x, W, g, b):
    C = x.shape[1]
    CP = na.shape[2]
    s = (g / jnp.sqrt(1.0 + EPS)).reshape(1, 64)
    bb = b.reshape(1, 64)
    return pl.pallas_call(
        _edge_kernel,
        grid=(N // BE,),
        in_specs=[
            pl.BlockSpec((BE, 16, CP), lambda i: (i, 0, 0)),
            pl.BlockSpec((BE, 16, CP), lambda i: (i, 0, 0)),
            pl.BlockSpec((BE, C), lambda i: (i, 0)),
            pl.BlockSpec((2 * C, 64), lambda i: (0, 0)),
            pl.BlockSpec((1, 64), lambda i: (0, 0)),
            pl.BlockSpec((1, 64), lambda i: (0, 0)),
        ],
        out_specs=pl.BlockSpec((BE, 64), lambda i: (i, 0)),
        out_shape=jax.ShapeDtypeStruct((N, 64), jnp.float32),
    )(na, nb, x, W.T, s, bb)


def _edge_block(x, W, g, b):
    C = x.shape[1]
    CP = 128
    x_pad = jnp.full((NP, CP), PADVAL, jnp.float32)
    x_pad = x_pad.at[:, C:].set(0.0)
    x_pad = x_pad.at[:N, :C].set(x)
    idx = _dist_topk(x_pad).T
    idx_flat = idx[:NPAD2].reshape(NPAD2 * KP)
    na, nb = _make_sc_gather(CP)(idx_flat, x_pad)
    return _edge_conv(na, nb, x, W, g, b)


def _head_kernel(xc_ref, w5_ref, s5_ref, b5_ref, wf_ref, sf_ref, bf_ref,
                 ws_ref, bs_ref, o_ref):
    y = jnp.dot(xc_ref[...], w5_ref[...], preferred_element_type=jnp.float32)
    y = _lrelu(y * s5_ref[...] + b5_ref[...])
    y = jnp.dot(y, wf_ref[...], preferred_element_type=jnp.float32)
    y = y * sf_ref[...] + bf_ref[...]
    y = jnp.dot(y, ws_ref[...], preferred_element_type=jnp.float32)
    o_ref[...] = y + bs_ref[...]


def _head(xcat, W5, g5, b5, Wf, gout, bout, Ws, bs):
    s5 = (g5 / jnp.sqrt(1.0 + EPS)).reshape(1, 1024)
    sf = (gout / jnp.sqrt(1.0 + EPS)).reshape(1, 256)
    return pl.pallas_call(
        _head_kernel,
        grid=(N // BE,),
        in_specs=[
            pl.BlockSpec((BE, 256), lambda i: (i, 0)),
            pl.BlockSpec((256, 1024), lambda i: (0, 0)),
            pl.BlockSpec((1, 1024), lambda i: (0, 0)),
            pl.BlockSpec((1, 1024), lambda i: (0, 0)),
            pl.BlockSpec((1024, 256), lambda i: (0, 0)),
            pl.BlockSpec((1, 256), lambda i: (0, 0)),
            pl.BlockSpec((1, 256), lambda i: (0, 0)),
            pl.BlockSpec((256, 20), lambda i: (0, 0)),
            pl.BlockSpec((1, 20), lambda i: (0, 0)),
        ],
        out_specs=pl.BlockSpec((BE, 20), lambda i: (i, 0)),
        out_shape=jax.ShapeDtypeStruct((N, 20), jnp.float32),
    )(xcat, W5.T, s5, b5.reshape(1, 1024), Wf.T, sf, bout.reshape(1, 256),
      Ws.T, bs.reshape(1, 20))


def kernel(points, features, W1, g1, b1, W2, g2, b2, W3, g3, b3, W4, g4, b4, W5, g5, b5, Wf, gout, bout, Ws, bs):
    x = jnp.concatenate([points, features], axis=1)
    x1 = _edge_block(x, W1, g1, b1)
    x2 = _edge_block(x1, W2, g2, b2)
    x3 = _edge_block(x2, W3, g3, b3)
    x4 = _edge_block(x3, W4, g4, b4)
    xcat = jnp.concatenate([x1, x2, x3, x4], axis=1)
    return _head(xcat, W5, g5, b5, Wf, gout, bout, Ws, bs)
